# Initial kernel scaffold; baseline (speedup 1.0000x reference)
import jax
import jax.numpy as jnp
from jax.experimental import pallas as pl


def kernel(x, vals_s, vals_v, W_s, b_s, W0, b0, src_s, dst_s, src_v, dst_v):
    raise NotImplementedError("write your pallas kernel here")


# speedup vs baseline: 5.0299x; 5.0299x over previous
import functools

import jax
import jax.numpy as jnp
from jax import lax
from jax.experimental import pallas as pl
from jax.experimental.pallas import tpu as pltpu
from jax.experimental.pallas import tpu_sc as plsc

N_VERT = 50000
K = 64
NC, NS = 2, 16
NW = NC * NS
R = 1568
NVP = NW * R
C_S = 512
C_V = 384
F = K // 16
NWP = NW + 16


def _mm_support(x_ref, w_ref, o_ref):
    o_ref[...] = jnp.dot(x_ref[...], w_ref[...], preferred_element_type=jnp.float32)


def _mm_hidden(h_ref, bs_ref, w_ref, o_ref):
    h = jnp.maximum(h_ref[...] + bs_ref[...], 0.0)
    o_ref[...] = jnp.dot(h, w_ref[...], preferred_element_type=jnp.float32)


def _sc_mesh():
    return plsc.VectorSubcoreMesh(core_axis_name="c", subcore_axis_name="s",
                                  num_cores=NC, num_subcores=NS)


_SC_PARAMS = pltpu.CompilerParams(use_tc_tiling_on_sc=False)


def _bounds(stv, env, wid):
    start = stv[pl.ds(wid, 16)][0]
    end = env[pl.ds(wid, 16)][0]
    return start, end


def _spmm_sensor(support, src, dst, vals, starts, ends, n_sens):
    @functools.partial(
        pl.kernel,
        out_type=jax.ShapeDtypeStruct((NVP, K), jnp.float32),
        mesh=_sc_mesh(),
        compiler_params=_SC_PARAMS,
        scratch_types=[
            pltpu.VMEM((R, K), jnp.float32),
            pltpu.VMEM((n_sens, K), jnp.float32),
            pltpu.VMEM((C_S,), jnp.int32),
            pltpu.VMEM((C_S,), jnp.int32),
            pltpu.VMEM((C_S,), jnp.float32),
            pltpu.VMEM((NWP,), jnp.int32),
            pltpu.VMEM((NWP,), jnp.int32),
        ],
    )
    def k(tab_hbm, src_hbm, dst_hbm, vals_hbm, st_hbm, en_hbm, out_hbm,
          acc, tab, srcv, dstv, valsv, stv, env):
        wid = lax.axis_index("s") * NC + lax.axis_index("c")
        base = wid * R
        pltpu.sync_copy(tab_hbm, tab)
        pltpu.sync_copy(st_hbm, stv)
        pltpu.sync_copy(en_hbm, env)
        start, end = _bounds(stv, env, wid)

        zero = jnp.zeros((16,), jnp.float32)

        def zrow(r, _):
            for f in range(F):
                acc[r, pl.ds(f * 16, 16)] = zero
            return 0
        lax.fori_loop(0, R, zrow, 0)

        astart = (start // C_S) * C_S
        nchunks = lax.max((end - astart + C_S - 1) // C_S, 0)

        def chunk(ci, _):
            cbase = astart + ci * C_S
            pltpu.sync_copy(src_hbm.at[pl.ds(cbase, C_S)], srcv)
            pltpu.sync_copy(dst_hbm.at[pl.ds(cbase, C_S)], dstv)
            pltpu.sync_copy(vals_hbm.at[pl.ds(cbase, C_S)], valsv)

            def group(gi, _):
                gb = gi * 16
                sv = srcv[pl.ds(gb, 16)]
                dlv = jnp.clip(dstv[pl.ds(gb, 16)] - base, 0, R - 1)
                ev = cbase + gb + lax.iota(jnp.int32, 16)
                okv = (ev >= start) & (ev < end)
                vv = jnp.where(okv, valsv[pl.ds(gb, 16)], 0.0)
                for l in range(16):
                    s = sv[l]
                    dl = dlv[l]
                    v = vv[l]
                    for f in range(F):
                        sl = pl.ds(f * 16, 16)
                        plsc.addupdate(acc.at[dl, sl], tab[s, sl] * v)
                return 0
            lax.fori_loop(0, C_S // 16, group, 0)
            return 0
        lax.fori_loop(0, nchunks, chunk, 0)
        pltpu.sync_copy(acc, out_hbm.at[pl.ds(base, R)])

    return k(support, src, dst, vals, starts, ends)


def _spmm_vertex(g, src2d, dst, vals, b0, starts, ends):
    @functools.partial(
        pl.kernel,
        out_type=jax.ShapeDtypeStruct((NVP, K), jnp.float32),
        mesh=_sc_mesh(),
        compiler_params=_SC_PARAMS,
        scratch_types=[
            pltpu.VMEM((R, K), jnp.float32),
            pltpu.VMEM((C_V // 128, 128), jnp.int32),
            pltpu.VMEM((C_V,), jnp.int32),
            pltpu.VMEM((C_V,), jnp.float32),
            pltpu.VMEM((C_V, K), jnp.float32),
            pltpu.VMEM((K,), jnp.float32),
            pltpu.VMEM((NWP,), jnp.int32),
            pltpu.VMEM((NWP,), jnp.int32),
            pltpu.SemaphoreType.DMA,
        ],
    )
    def k(g_hbm, src2_hbm, dst_hbm, vals_hbm, b0_hbm, st_hbm, en_hbm, out_hbm,
          acc, idx2, dstv, valsv, rows, b0v, stv, env, sem):
        wid = lax.axis_index("s") * NC + lax.axis_index("c")
        base = wid * R
        pltpu.sync_copy(st_hbm, stv)
        pltpu.sync_copy(en_hbm, env)
        pltpu.sync_copy(b0_hbm, b0v)
        start, end = _bounds(stv, env, wid)

        bvec = [b0v[pl.ds(f * 16, 16)] for f in range(F)]

        def brow(r, _):
            for f in range(F):
                acc[r, pl.ds(f * 16, 16)] = bvec[f]
            return 0
        lax.fori_loop(0, R, brow, 0)

        astart = (start // C_V) * C_V
        nchunks = lax.max((end - astart + C_V - 1) // C_V, 0)

        def chunk(ci, _):
            cbase = astart + ci * C_V
            pltpu.sync_copy(src2_hbm.at[pl.ds(cbase // 128, C_V // 128)], idx2)
            pltpu.sync_copy(dst_hbm.at[pl.ds(cbase, C_V)], dstv)
            pltpu.sync_copy(vals_hbm.at[pl.ds(cbase, C_V)], valsv)
            cps = [pltpu.async_copy(g_hbm.at[idx2.at[j]],
                                    rows.at[pl.ds(j * 128, 128)], sem)
                   for j in range(C_V // 128)]
            for cp in cps:
                cp.wait()

            def group(gi, _):
                gb = gi * 16
                dlv = jnp.clip(dstv[pl.ds(gb, 16)] - base, 0, R - 1)
                ev = cbase + gb + lax.iota(jnp.int32, 16)
                okv = (ev >= start) & (ev < end)
                vv = jnp.where(okv, valsv[pl.ds(gb, 16)], 0.0)
                for l in range(16):
                    el = gb + l
                    dl = dlv[l]
                    v = vv[l]
                    for f in range(F):
                        sl = pl.ds(f * 16, 16)
                        plsc.addupdate(acc.at[dl, sl], rows[el, sl] * v)
                return 0
            lax.fori_loop(0, C_V // 16, group, 0)
            return 0
        lax.fori_loop(0, nchunks, chunk, 0)
        pltpu.sync_copy(acc, out_hbm.at[pl.ds(base, R)])

    return k(g, src2d, dst, vals, b0, starts, ends)


def _pad_edges(src, dst, vals, chunk):
    nnz = src.shape[0]
    nnz_pad = ((nnz + chunk - 1) // chunk) * chunk
    pad = nnz_pad - nnz
    src = jnp.pad(src, (0, pad))
    dst = jnp.pad(dst, (0, pad), constant_values=NVP - 1)
    vals = jnp.pad(vals, (0, pad))
    bounds = (jnp.arange(NW, dtype=jnp.int32) * R).astype(dst.dtype)
    starts = jnp.searchsorted(dst, bounds, side="left").astype(jnp.int32)
    ends = jnp.concatenate(
        [starts[1:], jnp.array([nnz_pad], dtype=jnp.int32)])
    starts = jnp.pad(starts, (0, NWP - NW))
    ends = jnp.pad(ends, (0, NWP - NW))
    return src, dst, vals, starts, ends


def kernel(x, vals_s, vals_v, W_s, b_s, W0, b0, src_s, dst_s, src_v, dst_v):
    n_sens, k = x.shape
    l0 = W_s.shape[1]

    support = pl.pallas_call(
        _mm_support,
        out_shape=jax.ShapeDtypeStruct((n_sens, l0), jnp.float32),
    )(x, W_s)

    src_sp, dst_sp, vals_sp, starts_s, ends_s = _pad_edges(
        src_s, dst_s, vals_s, C_S)
    hpre = _spmm_sensor(support, src_sp, dst_sp, vals_sp,
                        starts_s, ends_s, n_sens)

    BLK = 1568
    g = pl.pallas_call(
        _mm_hidden,
        grid=(NVP // BLK,),
        in_specs=[
            pl.BlockSpec((BLK, l0), lambda i: (i, 0)),
            pl.BlockSpec((1, l0), lambda i: (0, 0)),
            pl.BlockSpec((l0, k), lambda i: (0, 0)),
        ],
        out_specs=pl.BlockSpec((BLK, k), lambda i: (i, 0)),
        out_shape=jax.ShapeDtypeStruct((NVP, k), jnp.float32),
    )(hpre, b_s.reshape(1, l0), W0)

    src_vp, dst_vp, vals_vp, starts_v, ends_v = _pad_edges(
        src_v, dst_v, vals_v, C_V)
    src2d = src_vp.reshape(-1, 128)
    out = _spmm_vertex(g, src2d, dst_vp, vals_vp, b0, starts_v, ends_v)
    return out[:N_VERT]
